# baseline (speedup 1.0000x reference)
import functools

import jax
import jax.numpy as jnp
from jax import lax
from jax.experimental import pallas as pl
from jax.experimental.pallas import tpu as pltpu
from jax.experimental.pallas import tpu_sc as plsc

_NC = 2
_NS = 16
_LANES = 16


def _mm_first(x, W):
    N, K = x.shape
    M = W.shape[1]
    BN = 1000

    def body(x_ref, w_ref, o_ref):
        o_ref[...] = jnp.dot(x_ref[...], w_ref[...],
                             preferred_element_type=jnp.float32)

    return pl.pallas_call(
        body,
        grid=(N // BN,),
        in_specs=[
            pl.BlockSpec((BN, K), lambda i: (i, 0)),
            pl.BlockSpec((K, M), lambda i: (0, 0)),
        ],
        out_specs=pl.BlockSpec((BN, M), lambda i: (i, 0)),
        out_shape=jax.ShapeDtypeStruct((N, M), jnp.float32),
    )(x, W)


def _mm_fused(parts, b, W):
    _, _, N, H = parts.shape
    K = 2 * H
    M = W.shape[1]
    BN = 1000

    def body(p_ref, b_ref, w_ref, o_ref):
        h0 = p_ref[0, 0] + p_ref[1, 0]
        h1 = p_ref[0, 1] + p_ref[1, 1]
        h = jnp.concatenate([h0, h1], axis=-1)
        h = jnp.maximum(h + b_ref[...], 0.0)
        o_ref[...] = jnp.dot(h, w_ref[...],
                             preferred_element_type=jnp.float32)

    return pl.pallas_call(
        body,
        grid=(N // BN,),
        in_specs=[
            pl.BlockSpec((2, 2, BN, H), lambda i: (0, 0, i, 0)),
            pl.BlockSpec((K,), lambda i: (0,)),
            pl.BlockSpec((K, M), lambda i: (0, 0)),
        ],
        out_specs=pl.BlockSpec((BN, M), lambda i: (i, 0)),
        out_shape=jax.ShapeDtypeStruct((N, M), jnp.float32),
    )(parts, b, W)


def _final_act(parts, b):
    _, _, N, H = parts.shape
    K = 2 * H
    BN = 1000

    def body(p_ref, b_ref, o_ref):
        h0 = p_ref[0, 0] + p_ref[1, 0]
        h1 = p_ref[0, 1] + p_ref[1, 1]
        h = jnp.concatenate([h0, h1], axis=-1)
        o_ref[...] = jnp.maximum(h + b_ref[...], 0.0)

    return pl.pallas_call(
        body,
        grid=(N // BN,),
        in_specs=[
            pl.BlockSpec((2, 2, BN, H), lambda i: (0, 0, i, 0)),
            pl.BlockSpec((K,), lambda i: (0,)),
        ],
        out_specs=pl.BlockSpec((BN, K), lambda i: (i, 0)),
        out_shape=jax.ShapeDtypeStruct((N, K), jnp.float32),
    )(parts, b)


def _sc_aggregate(h2p, packed2, w2, zeros):
    _, NP, PW = h2p.shape
    NW, e_per_w = packed2.shape
    CH = 32
    n32 = e_per_w // CH
    HF = PW // 2
    R = (NP // _NS) & ~7
    TAIL = NP - _NS * R

    mesh = plsc.VectorSubcoreMesh(core_axis_name="c", subcore_axis_name="s",
                                  num_cores=_NC, num_subcores=_NS)

    @functools.partial(
        pl.kernel,
        mesh=mesh,
        out_type=jax.ShapeDtypeStruct((_NC, 2, NP, PW), jnp.float32),
        compiler_params=pltpu.CompilerParams(needs_layout_passes=False),
        scratch_types=[
            pltpu.VMEM((e_per_w,), jnp.int32),
            pltpu.VMEM((e_per_w,), jnp.float32),
            pltpu.VMEM((CH, PW), jnp.float32),
            pltpu.VMEM((CH, PW), jnp.float32),
            pltpu.VMEM((CH, PW), jnp.float32),
            pltpu.VMEM((CH, PW), jnp.float32),
            pltpu.VMEM((CH,), jnp.int32),
            pltpu.VMEM((CH,), jnp.int32),
            pltpu.VMEM((CH,), jnp.int32),
            pltpu.VMEM((CH,), jnp.int32),
            pltpu.VMEM_SHARED((NP, PW), jnp.float32),
            pltpu.VMEM_SHARED((NP, PW), jnp.float32),
            pltpu.SemaphoreType.DMA,
            pltpu.SemaphoreType.DMA,
            pltpu.SemaphoreType.DMA,
            pltpu.SemaphoreType.DMA,
        ],
    )
    def k(h_hbm, pk_hbm, w_hbm, z_hbm, out_hbm,
          pk_all, w_all, gbuf0, gbuf1, sbuf0, sbuf1,
          ibuf0, ibuf1, dbuf0, dbuf1, hsp, acc_sp,
          gsem0, gsem1, ssem0, ssem1):
        c = lax.axis_index("c")
        s = lax.axis_index("s")
        wid = s * _NC + c
        gbuf = (gbuf0, gbuf1)
        sbuf = (sbuf0, sbuf1)
        ibuf = (ibuf0, ibuf1)
        dbuf = (dbuf0, dbuf1)
        gsem = (gsem0, gsem1)
        ssem = (ssem0, ssem1)
        iot = lax.iota(jnp.int32, 16)
        zero16 = jnp.zeros((16,), jnp.float32)

        pltpu.sync_copy(pk_hbm.at[wid], pk_all)
        pltpu.sync_copy(w_hbm.at[wid], w_all)

        def stage_gather_idx(b, ci):
            for v in range(CH // _LANES):
                pk = pk_all[pl.ds(ci * CH + v * _LANES, _LANES)]
                ibuf[b][pl.ds(v * _LANES, _LANES)] = (
                    lax.shift_right_logical(pk, 17))

        for half in range(2):
            pltpu.sync_copy(h_hbm.at[half, pl.ds(s * R, R)],
                            hsp.at[pl.ds(s * R, R)])
            pltpu.sync_copy(z_hbm.at[pl.ds(s * R, R)],
                            acc_sp.at[pl.ds(s * R, R)])

            @pl.when(s == 0)
            def _():
                pltpu.sync_copy(h_hbm.at[half, pl.ds(_NS * R, TAIL)],
                                hsp.at[pl.ds(_NS * R, TAIL)])
                pltpu.sync_copy(z_hbm.at[pl.ds(_NS * R, TAIL)],
                                acc_sp.at[pl.ds(_NS * R, TAIL)])

            plsc.subcore_barrier()

            for b in range(2):
                stage_gather_idx(b, b)
                pltpu.async_copy(hsp.at[ibuf[b]], gbuf[b], gsem[b])

            @pl.loop(0, n32, step=2)
            def _(i):
                for b in range(2):
                    ci = i + b
                    pltpu.make_async_copy(hsp.at[ibuf[b]],
                                          gbuf[b], gsem[b]).wait()

                    @pl.when(ci >= 2)
                    def _():
                        pltpu.make_async_copy(
                            sbuf[b], acc_sp.at[dbuf[b]], ssem[b]).wait()

                    for v in range(CH // _LANES):
                        pk = pk_all[pl.ds(ci * CH + v * _LANES, _LANES)]
                        dbuf[b][pl.ds(v * _LANES, _LANES)] = (
                            lax.shift_right_logical(
                                jnp.bitwise_and(pk, 0xFFFF), 1))

                    base = jnp.full((_LANES,), ci * CH, jnp.int32)

                    @plsc.parallel_loop(0, CH, unroll=4)
                    def _(e):
                        idx = base + e
                        pb = plsc.load_gather(pk_all, [idx])
                        wb = plsc.load_gather(w_all, [idx])
                        sofs = jnp.bitwise_and(
                            lax.shift_right_logical(pb, 16), 1) * HF
                        dofs = jnp.bitwise_and(pb, 1) * HF
                        nofs = HF - dofs
                        erow = jnp.full((_LANES,), e, jnp.int32)
                        for j in range(HF // _LANES):
                            ln = j * _LANES + iot
                            v = plsc.load_gather(gbuf[b],
                                                 [erow, sofs + ln])
                            plsc.store_scatter(sbuf[b],
                                               [erow, dofs + ln], v * wb)
                            plsc.store_scatter(sbuf[b],
                                               [erow, nofs + ln], zero16)

                    @pl.when(ci + 2 < n32)
                    def _():
                        stage_gather_idx(b, ci + 2)
                        pltpu.async_copy(hsp.at[ibuf[b]], gbuf[b], gsem[b])

                    pltpu.async_copy(sbuf[b], acc_sp.at[dbuf[b]],
                                     ssem[b], add=True)

            for b in range(2):
                pltpu.make_async_copy(sbuf[b], acc_sp.at[dbuf[b]],
                                      ssem[b]).wait()
            plsc.subcore_barrier()

            pltpu.sync_copy(acc_sp.at[pl.ds(s * R, R)],
                            out_hbm.at[c, half, pl.ds(s * R, R)])

            @pl.when(s == 0)
            def _():
                pltpu.sync_copy(
                    acc_sp.at[pl.ds(_NS * R, TAIL)],
                    out_hbm.at[c, half, pl.ds(_NS * R, TAIL)])

    return k(h2p, packed2, w2, zeros)


def _pair_halves(y):
    N = y.shape[0]
    return jnp.stack([y[:, :64].reshape(N // 2, 128),
                      y[:, 64:].reshape(N // 2, 128)])


def _unpair(parts):
    nc, nh, NP, PW = parts.shape
    return parts.reshape(nc, nh, NP * 2, PW // 2)


def kernel(x, adj_index, adj_weight, W1, b1, W2, b2, W3, b3):
    src = adj_index[0].astype(jnp.int32)
    dst = adj_index[1].astype(jnp.int32)
    w = adj_weight.astype(jnp.float32)
    N, _ = x.shape
    zeros = jnp.zeros((N // 2, 128), jnp.float32)

    E = src.shape[0]
    NW = _NC * _NS
    grain = NW * 32 * 2
    E_pad = ((E + grain - 1) // grain) * grain
    if E_pad != E:
        pad = E_pad - E
        src = jnp.pad(src, (0, pad))
        dst = jnp.pad(dst, (0, pad))
        w = jnp.pad(w, (0, pad))
    packed2 = ((src << 16) | dst).reshape(NW, E_pad // NW)
    w2 = w.reshape(NW, E_pad // NW)

    y = _mm_first(x, W1)
    p = _unpair(_sc_aggregate(_pair_halves(y), packed2, w2, zeros))
    y = _mm_fused(p, b1, W2)
    p = _unpair(_sc_aggregate(_pair_halves(y), packed2, w2, zeros))
    y = _mm_fused(p, b2, W3)
    p = _unpair(_sc_aggregate(_pair_halves(y), packed2, w2, zeros))
    return _final_act(p, b3)


# speedup vs baseline: 1.0151x; 1.0151x over previous
import functools

import jax
import jax.numpy as jnp
from jax import lax
from jax.experimental import pallas as pl
from jax.experimental.pallas import tpu as pltpu
from jax.experimental.pallas import tpu_sc as plsc

_NC = 2
_NS = 16
_LANES = 16


def _mm_first(x, W):
    N, K = x.shape
    M = W.shape[1]
    BN = 1000

    def body(x_ref, w_ref, o_ref):
        o_ref[...] = jnp.dot(x_ref[...], w_ref[...],
                             preferred_element_type=jnp.float32)

    return pl.pallas_call(
        body,
        grid=(N // BN,),
        in_specs=[
            pl.BlockSpec((BN, K), lambda i: (i, 0)),
            pl.BlockSpec((K, M), lambda i: (0, 0)),
        ],
        out_specs=pl.BlockSpec((BN, M), lambda i: (i, 0)),
        out_shape=jax.ShapeDtypeStruct((N, M), jnp.float32),
    )(x, W)


def _mm_fused(parts, b, W):
    _, _, N, H = parts.shape
    K = 2 * H
    M = W.shape[1]
    BN = 1000

    def body(p_ref, b_ref, w_ref, o_ref):
        h0 = p_ref[0, 0] + p_ref[1, 0]
        h1 = p_ref[0, 1] + p_ref[1, 1]
        h = jnp.concatenate([h0, h1], axis=-1)
        h = jnp.maximum(h + b_ref[...], 0.0)
        o_ref[...] = jnp.dot(h, w_ref[...],
                             preferred_element_type=jnp.float32)

    return pl.pallas_call(
        body,
        grid=(N // BN,),
        in_specs=[
            pl.BlockSpec((2, 2, BN, H), lambda i: (0, 0, i, 0)),
            pl.BlockSpec((K,), lambda i: (0,)),
            pl.BlockSpec((K, M), lambda i: (0, 0)),
        ],
        out_specs=pl.BlockSpec((BN, M), lambda i: (i, 0)),
        out_shape=jax.ShapeDtypeStruct((N, M), jnp.float32),
    )(parts, b, W)


def _final_act(parts, b):
    _, _, N, H = parts.shape
    K = 2 * H
    BN = 1000

    def body(p_ref, b_ref, o_ref):
        h0 = p_ref[0, 0] + p_ref[1, 0]
        h1 = p_ref[0, 1] + p_ref[1, 1]
        h = jnp.concatenate([h0, h1], axis=-1)
        o_ref[...] = jnp.maximum(h + b_ref[...], 0.0)

    return pl.pallas_call(
        body,
        grid=(N // BN,),
        in_specs=[
            pl.BlockSpec((2, 2, BN, H), lambda i: (0, 0, i, 0)),
            pl.BlockSpec((K,), lambda i: (0,)),
        ],
        out_specs=pl.BlockSpec((BN, K), lambda i: (i, 0)),
        out_shape=jax.ShapeDtypeStruct((N, K), jnp.float32),
    )(parts, b)


def _sc_aggregate(h2p, packed2, w2, zeros):
    _, NP, PW = h2p.shape
    NW, e_per_w = packed2.shape
    CH = 48
    n32 = e_per_w // CH
    HF = PW // 2
    R = (NP // _NS) & ~7
    TAIL = NP - _NS * R

    mesh = plsc.VectorSubcoreMesh(core_axis_name="c", subcore_axis_name="s",
                                  num_cores=_NC, num_subcores=_NS)

    @functools.partial(
        pl.kernel,
        mesh=mesh,
        out_type=jax.ShapeDtypeStruct((_NC, 2, NP, PW), jnp.float32),
        compiler_params=pltpu.CompilerParams(needs_layout_passes=False),
        scratch_types=[
            pltpu.VMEM((e_per_w,), jnp.int32),
            pltpu.VMEM((e_per_w,), jnp.float32),
            pltpu.VMEM((CH, PW), jnp.float32),
            pltpu.VMEM((CH, PW), jnp.float32),
            pltpu.VMEM((CH, PW), jnp.float32),
            pltpu.VMEM((CH, PW), jnp.float32),
            pltpu.VMEM((CH,), jnp.int32),
            pltpu.VMEM((CH,), jnp.int32),
            pltpu.VMEM((CH,), jnp.int32),
            pltpu.VMEM((CH,), jnp.int32),
            pltpu.VMEM_SHARED((NP, PW), jnp.float32),
            pltpu.VMEM_SHARED((NP, PW), jnp.float32),
            pltpu.SemaphoreType.DMA,
            pltpu.SemaphoreType.DMA,
            pltpu.SemaphoreType.DMA,
            pltpu.SemaphoreType.DMA,
        ],
    )
    def k(h_hbm, pk_hbm, w_hbm, z_hbm, out_hbm,
          pk_all, w_all, gbuf0, gbuf1, sbuf0, sbuf1,
          ibuf0, ibuf1, dbuf0, dbuf1, hsp, acc_sp,
          gsem0, gsem1, ssem0, ssem1):
        c = lax.axis_index("c")
        s = lax.axis_index("s")
        wid = s * _NC + c
        gbuf = (gbuf0, gbuf1)
        sbuf = (sbuf0, sbuf1)
        ibuf = (ibuf0, ibuf1)
        dbuf = (dbuf0, dbuf1)
        gsem = (gsem0, gsem1)
        ssem = (ssem0, ssem1)
        iot = lax.iota(jnp.int32, 16)
        zero16 = jnp.zeros((16,), jnp.float32)

        pltpu.sync_copy(pk_hbm.at[wid], pk_all)
        pltpu.sync_copy(w_hbm.at[wid], w_all)

        def stage_gather_idx(b, ci):
            for v in range(CH // _LANES):
                pk = pk_all[pl.ds(ci * CH + v * _LANES, _LANES)]
                ibuf[b][pl.ds(v * _LANES, _LANES)] = (
                    lax.shift_right_logical(pk, 17))

        for half in range(2):
            pltpu.sync_copy(h_hbm.at[half, pl.ds(s * R, R)],
                            hsp.at[pl.ds(s * R, R)])
            pltpu.sync_copy(z_hbm.at[pl.ds(s * R, R)],
                            acc_sp.at[pl.ds(s * R, R)])

            @pl.when(s == 0)
            def _():
                pltpu.sync_copy(h_hbm.at[half, pl.ds(_NS * R, TAIL)],
                                hsp.at[pl.ds(_NS * R, TAIL)])
                pltpu.sync_copy(z_hbm.at[pl.ds(_NS * R, TAIL)],
                                acc_sp.at[pl.ds(_NS * R, TAIL)])

            plsc.subcore_barrier()

            for b in range(2):
                stage_gather_idx(b, b)
                pltpu.async_copy(hsp.at[ibuf[b]], gbuf[b], gsem[b])

            @pl.loop(0, n32, step=2)
            def _(i):
                for b in range(2):
                    ci = i + b
                    pltpu.make_async_copy(hsp.at[ibuf[b]],
                                          gbuf[b], gsem[b]).wait()

                    @pl.when(ci >= 2)
                    def _():
                        pltpu.make_async_copy(
                            sbuf[b], acc_sp.at[dbuf[b]], ssem[b]).wait()

                    for v in range(CH // _LANES):
                        pk = pk_all[pl.ds(ci * CH + v * _LANES, _LANES)]
                        dbuf[b][pl.ds(v * _LANES, _LANES)] = (
                            lax.shift_right_logical(
                                jnp.bitwise_and(pk, 0xFFFF), 1))

                    base = jnp.full((_LANES,), ci * CH, jnp.int32)

                    @plsc.parallel_loop(0, CH, unroll=4)
                    def _(e):
                        idx = base + e
                        pb = plsc.load_gather(pk_all, [idx])
                        wb = plsc.load_gather(w_all, [idx])
                        sofs = jnp.bitwise_and(
                            lax.shift_right_logical(pb, 16), 1) * HF
                        dofs = jnp.bitwise_and(pb, 1) * HF
                        nofs = HF - dofs
                        erow = jnp.full((_LANES,), e, jnp.int32)
                        for j in range(HF // _LANES):
                            ln = j * _LANES + iot
                            v = plsc.load_gather(gbuf[b],
                                                 [erow, sofs + ln])
                            plsc.store_scatter(sbuf[b],
                                               [erow, dofs + ln], v * wb)
                            plsc.store_scatter(sbuf[b],
                                               [erow, nofs + ln], zero16)

                    @pl.when(ci + 2 < n32)
                    def _():
                        stage_gather_idx(b, ci + 2)
                        pltpu.async_copy(hsp.at[ibuf[b]], gbuf[b], gsem[b])

                    pltpu.async_copy(sbuf[b], acc_sp.at[dbuf[b]],
                                     ssem[b], add=True)

            for b in range(2):
                pltpu.make_async_copy(sbuf[b], acc_sp.at[dbuf[b]],
                                      ssem[b]).wait()
            plsc.subcore_barrier()

            pltpu.sync_copy(acc_sp.at[pl.ds(s * R, R)],
                            out_hbm.at[c, half, pl.ds(s * R, R)])

            @pl.when(s == 0)
            def _():
                pltpu.sync_copy(
                    acc_sp.at[pl.ds(_NS * R, TAIL)],
                    out_hbm.at[c, half, pl.ds(_NS * R, TAIL)])

    return k(h2p, packed2, w2, zeros)


def _pair_halves(y):
    N = y.shape[0]
    return jnp.stack([y[:, :64].reshape(N // 2, 128),
                      y[:, 64:].reshape(N // 2, 128)])


def _unpair(parts):
    nc, nh, NP, PW = parts.shape
    return parts.reshape(nc, nh, NP * 2, PW // 2)


def kernel(x, adj_index, adj_weight, W1, b1, W2, b2, W3, b3):
    src = adj_index[0].astype(jnp.int32)
    dst = adj_index[1].astype(jnp.int32)
    w = adj_weight.astype(jnp.float32)
    N, _ = x.shape
    zeros = jnp.zeros((N // 2, 128), jnp.float32)

    E = src.shape[0]
    NW = _NC * _NS
    grain = NW * 48 * 2
    E_pad = ((E + grain - 1) // grain) * grain
    if E_pad != E:
        pad = E_pad - E
        src = jnp.pad(src, (0, pad))
        dst = jnp.pad(dst, (0, pad))
        w = jnp.pad(w, (0, pad))
    packed2 = ((src << 16) | dst).reshape(NW, E_pad // NW)
    w2 = w.reshape(NW, E_pad // NW)

    y = _mm_first(x, W1)
    p = _unpair(_sc_aggregate(_pair_halves(y), packed2, w2, zeros))
    y = _mm_fused(p, b1, W2)
    p = _unpair(_sc_aggregate(_pair_halves(y), packed2, w2, zeros))
    y = _mm_fused(p, b2, W3)
    p = _unpair(_sc_aggregate(_pair_halves(y), packed2, w2, zeros))
    return _final_act(p, b3)


# speedup vs baseline: 1.0331x; 1.0178x over previous
import functools

import jax
import jax.numpy as jnp
from jax import lax
from jax.experimental import pallas as pl
from jax.experimental.pallas import tpu as pltpu
from jax.experimental.pallas import tpu_sc as plsc

_NC = 2
_NS = 16
_LANES = 16


def _mm_first(x, W):
    N, K = x.shape
    M = W.shape[1]
    BN = 1000

    def body(x_ref, w_ref, o_ref):
        o_ref[...] = jnp.dot(x_ref[...], w_ref[...],
                             preferred_element_type=jnp.float32)

    return pl.pallas_call(
        body,
        grid=(N // BN,),
        in_specs=[
            pl.BlockSpec((BN, K), lambda i: (i, 0)),
            pl.BlockSpec((K, M), lambda i: (0, 0)),
        ],
        out_specs=pl.BlockSpec((BN, M), lambda i: (i, 0)),
        out_shape=jax.ShapeDtypeStruct((N, M), jnp.float32),
    )(x, W)


def _mm_fused(parts, b, W):
    _, _, N, H = parts.shape
    K = 2 * H
    M = W.shape[1]
    BN = 1000

    def body(p_ref, b_ref, w_ref, o_ref):
        h0 = p_ref[0, 0] + p_ref[1, 0]
        h1 = p_ref[0, 1] + p_ref[1, 1]
        h = jnp.concatenate([h0, h1], axis=-1)
        h = jnp.maximum(h + b_ref[...], 0.0)
        o_ref[...] = jnp.dot(h, w_ref[...],
                             preferred_element_type=jnp.float32)

    return pl.pallas_call(
        body,
        grid=(N // BN,),
        in_specs=[
            pl.BlockSpec((2, 2, BN, H), lambda i: (0, 0, i, 0)),
            pl.BlockSpec((K,), lambda i: (0,)),
            pl.BlockSpec((K, M), lambda i: (0, 0)),
        ],
        out_specs=pl.BlockSpec((BN, M), lambda i: (i, 0)),
        out_shape=jax.ShapeDtypeStruct((N, M), jnp.float32),
    )(parts, b, W)


def _final_act(parts, b):
    _, _, N, H = parts.shape
    K = 2 * H
    BN = 1000

    def body(p_ref, b_ref, o_ref):
        h0 = p_ref[0, 0] + p_ref[1, 0]
        h1 = p_ref[0, 1] + p_ref[1, 1]
        h = jnp.concatenate([h0, h1], axis=-1)
        o_ref[...] = jnp.maximum(h + b_ref[...], 0.0)

    return pl.pallas_call(
        body,
        grid=(N // BN,),
        in_specs=[
            pl.BlockSpec((2, 2, BN, H), lambda i: (0, 0, i, 0)),
            pl.BlockSpec((K,), lambda i: (0,)),
        ],
        out_specs=pl.BlockSpec((BN, K), lambda i: (i, 0)),
        out_shape=jax.ShapeDtypeStruct((N, K), jnp.float32),
    )(parts, b)


def _sc_aggregate(h2p, packed2, w2, zeros):
    _, NP, PW = h2p.shape
    NW, e_per_w = packed2.shape
    CH = 48
    n32 = e_per_w // CH
    HF = PW // 2
    R = (NP // _NS) & ~7
    TAIL = NP - _NS * R

    mesh = plsc.VectorSubcoreMesh(core_axis_name="c", subcore_axis_name="s",
                                  num_cores=_NC, num_subcores=_NS)

    @functools.partial(
        pl.kernel,
        mesh=mesh,
        out_type=jax.ShapeDtypeStruct((_NC, 2, NP, PW), jnp.float32),
        compiler_params=pltpu.CompilerParams(needs_layout_passes=False),
        scratch_types=[
            pltpu.VMEM((e_per_w,), jnp.int32),
            pltpu.VMEM((e_per_w,), jnp.float32),
            pltpu.VMEM((CH, PW), jnp.float32),
            pltpu.VMEM((CH, PW), jnp.float32),
            pltpu.VMEM((CH, PW), jnp.float32),
            pltpu.VMEM((CH, PW), jnp.float32),
            pltpu.VMEM((CH,), jnp.int32),
            pltpu.VMEM((CH,), jnp.int32),
            pltpu.VMEM((CH,), jnp.int32),
            pltpu.VMEM((CH,), jnp.int32),
            pltpu.VMEM_SHARED((NP, PW), jnp.float32),
            pltpu.VMEM_SHARED((NP, PW), jnp.float32),
            pltpu.SemaphoreType.DMA,
            pltpu.SemaphoreType.DMA,
            pltpu.SemaphoreType.DMA,
            pltpu.SemaphoreType.DMA,
        ],
    )
    def k(h_hbm, pk_hbm, w_hbm, z_hbm, out_hbm,
          pk_all, w_all, gbuf0, gbuf1, sbuf0, sbuf1,
          ibuf0, ibuf1, dbuf0, dbuf1, hsp, acc_sp,
          gsem0, gsem1, ssem0, ssem1):
        c = lax.axis_index("c")
        s = lax.axis_index("s")
        wid = s * _NC + c
        gbuf = (gbuf0, gbuf1)
        sbuf = (sbuf0, sbuf1)
        ibuf = (ibuf0, ibuf1)
        dbuf = (dbuf0, dbuf1)
        gsem = (gsem0, gsem1)
        ssem = (ssem0, ssem1)
        iot = lax.iota(jnp.int32, 16)
        zero16 = jnp.zeros((16,), jnp.float32)

        pltpu.sync_copy(pk_hbm.at[wid], pk_all)
        pltpu.sync_copy(w_hbm.at[wid], w_all)

        def stage_gather_idx(b, ci):
            for v in range(CH // _LANES):
                pk = pk_all[pl.ds(ci * CH + v * _LANES, _LANES)]
                ibuf[b][pl.ds(v * _LANES, _LANES)] = (
                    lax.shift_right_logical(pk, 17))

        for half in range(2):
            pltpu.sync_copy(h_hbm.at[half, pl.ds(s * R, R)],
                            hsp.at[pl.ds(s * R, R)])
            pltpu.sync_copy(z_hbm.at[pl.ds(s * R, R)],
                            acc_sp.at[pl.ds(s * R, R)])

            @pl.when(s == 0)
            def _():
                pltpu.sync_copy(h_hbm.at[half, pl.ds(_NS * R, TAIL)],
                                hsp.at[pl.ds(_NS * R, TAIL)])
                pltpu.sync_copy(z_hbm.at[pl.ds(_NS * R, TAIL)],
                                acc_sp.at[pl.ds(_NS * R, TAIL)])

            plsc.subcore_barrier()

            for b in range(2):
                stage_gather_idx(b, b)
                pltpu.async_copy(hsp.at[ibuf[b]], gbuf[b], gsem[b])

            @pl.loop(0, n32, step=2)
            def _(i):
                for b in range(2):
                    ci = i + b
                    pltpu.make_async_copy(hsp.at[ibuf[b]],
                                          gbuf[b], gsem[b]).wait()

                    @pl.when(ci >= 2)
                    def _():
                        pltpu.make_async_copy(
                            sbuf[b], acc_sp.at[dbuf[b]], ssem[b]).wait()

                    for v in range(CH // _LANES):
                        pk = pk_all[pl.ds(ci * CH + v * _LANES, _LANES)]
                        dbuf[b][pl.ds(v * _LANES, _LANES)] = (
                            lax.shift_right_logical(
                                jnp.bitwise_and(pk, 0xFFFF), 1))

                    base = jnp.full((_LANES,), ci * CH, jnp.int32)

                    @plsc.parallel_loop(0, CH, unroll=8)
                    def _(e):
                        idx = base + e
                        pb = plsc.load_gather(pk_all, [idx])
                        wb = plsc.load_gather(w_all, [idx])
                        sofs = jnp.bitwise_and(
                            lax.shift_right_logical(pb, 16), 1) * HF
                        dofs = jnp.bitwise_and(pb, 1) * HF
                        nofs = HF - dofs
                        erow = jnp.full((_LANES,), e, jnp.int32)
                        for j in range(HF // _LANES):
                            ln = j * _LANES + iot
                            v = plsc.load_gather(gbuf[b],
                                                 [erow, sofs + ln])
                            plsc.store_scatter(sbuf[b],
                                               [erow, dofs + ln], v * wb)
                            plsc.store_scatter(sbuf[b],
                                               [erow, nofs + ln], zero16)

                    @pl.when(ci + 2 < n32)
                    def _():
                        stage_gather_idx(b, ci + 2)
                        pltpu.async_copy(hsp.at[ibuf[b]], gbuf[b], gsem[b])

                    pltpu.async_copy(sbuf[b], acc_sp.at[dbuf[b]],
                                     ssem[b], add=True)

            for b in range(2):
                pltpu.make_async_copy(sbuf[b], acc_sp.at[dbuf[b]],
                                      ssem[b]).wait()
            plsc.subcore_barrier()

            pltpu.sync_copy(acc_sp.at[pl.ds(s * R, R)],
                            out_hbm.at[c, half, pl.ds(s * R, R)])

            @pl.when(s == 0)
            def _():
                pltpu.sync_copy(
                    acc_sp.at[pl.ds(_NS * R, TAIL)],
                    out_hbm.at[c, half, pl.ds(_NS * R, TAIL)])

    return k(h2p, packed2, w2, zeros)


def _pair_halves(y):
    N = y.shape[0]
    return jnp.stack([y[:, :64].reshape(N // 2, 128),
                      y[:, 64:].reshape(N // 2, 128)])


def _unpair(parts):
    nc, nh, NP, PW = parts.shape
    return parts.reshape(nc, nh, NP * 2, PW // 2)


def kernel(x, adj_index, adj_weight, W1, b1, W2, b2, W3, b3):
    src = adj_index[0].astype(jnp.int32)
    dst = adj_index[1].astype(jnp.int32)
    w = adj_weight.astype(jnp.float32)
    N, _ = x.shape
    zeros = jnp.zeros((N // 2, 128), jnp.float32)

    E = src.shape[0]
    NW = _NC * _NS
    grain = NW * 48 * 2
    E_pad = ((E + grain - 1) // grain) * grain
    if E_pad != E:
        pad = E_pad - E
        src = jnp.pad(src, (0, pad))
        dst = jnp.pad(dst, (0, pad))
        w = jnp.pad(w, (0, pad))
    packed2 = ((src << 16) | dst).reshape(NW, E_pad // NW)
    w2 = w.reshape(NW, E_pad // NW)

    y = _mm_first(x, W1)
    p = _unpair(_sc_aggregate(_pair_halves(y), packed2, w2, zeros))
    y = _mm_fused(p, b1, W2)
    p = _unpair(_sc_aggregate(_pair_halves(y), packed2, w2, zeros))
    y = _mm_fused(p, b2, W3)
    p = _unpair(_sc_aggregate(_pair_halves(y), packed2, w2, zeros))
    return _final_act(p, b3)
